# baseline (speedup 1.0000x reference)
import functools

import jax
import jax.numpy as jnp
from jax import lax
from jax.experimental import pallas as pl
from jax.experimental.pallas import tpu as pltpu
from jax.experimental.pallas import tpu_sc as plsc

N = 10000
E = 160000
F_IN = 256
D = 512
BATCH = 100

Q = 4
DQ = D // Q
NS = 16
NPASS = 3
NT = 3456
NOUT = NPASS * NT
DUMP = NT
ACC_R = NT + 8
CH = 128
NCHUNK = 80
EPT = NCHUNK * CH
E_PAD = NS * EPT
PAD_DST = 1 << 30
NPT = 216
ZROWS = NPT

_VEC = 16


def _segsum_body(h4_hbm, src_hbm, dst_hbm, out_hbm,
                 src_v, dst_v, gidx_v, ldst_v, rows2, zbuf, sem_a, acc):
    c = lax.axis_index("c")
    s = lax.axis_index("s")

    pltpu.sync_copy(src_hbm.at[s], src_v)
    pltpu.sync_copy(dst_hbm.at[s], dst_v)

    zeros = jnp.zeros((_VEC,), jnp.float32)

    def zfill(i, carry):
        for kk in range(DQ // _VEC):
            zbuf[i, pl.ds(kk * _VEC, _VEC)] = zeros
        return carry

    lax.fori_loop(0, ZROWS, zfill, 0)

    dump16 = jnp.full((_VEC,), DUMP, jnp.int32)

    for q in range(Q):
        def gidx(i, carry):
            for kk in range(CH // _VEC):
                sl = pl.ds(kk * _VEC, _VEC)
                gidx_v[i, sl] = src_v[i, sl] * Q + q
            return carry

        lax.fori_loop(0, NCHUNK, gidx, 0)

        for t in range(NPASS):
            lo = t * NT
            lo16 = jnp.full((_VEC,), lo, jnp.int32)
            hi16 = jnp.full((_VEC,), lo + NT, jnp.int32)

            def localize(i, carry):
                for kk in range(CH // _VEC):
                    sl = pl.ds(kk * _VEC, _VEC)
                    dv = dst_v[i, sl]
                    oob = (dv < lo16) | (dv >= hi16)
                    ldst_v[i, sl] = jnp.where(oob, dump16, dv - lo16)
                return carry

            lax.fori_loop(0, NCHUNK, localize, 0)

            pltpu.sync_copy(zbuf, acc.at[pl.ds(s * NPT, ZROWS)])

            plsc.subcore_barrier()

            def chunk(j, carry):
                pltpu.async_copy(h4_hbm.at[gidx_v.at[j]], rows2.at[0],
                                 sem_a).wait()
                pltpu.sync_copy(rows2.at[0], acc.at[ldst_v.at[j]], add=True)
                return carry

            lax.fori_loop(0, NCHUNK, chunk, 0)

            plsc.subcore_barrier()

            pltpu.sync_copy(
                acc.at[pl.ds(s * NPT, NPT)],
                out_hbm.at[q].at[pl.ds(lo + s * NPT, NPT)])


def _build_segsum():
    mesh = plsc.VectorSubcoreMesh(core_axis_name="c", subcore_axis_name="s",
                                  num_cores=1)

    @functools.partial(
        pl.kernel,
        out_type=jax.ShapeDtypeStruct((Q, NOUT, DQ), jnp.float32),
        mesh=mesh,
        scratch_types=[
            pltpu.VMEM((NCHUNK, CH), jnp.int32),
            pltpu.VMEM((NCHUNK, CH), jnp.int32),
            pltpu.VMEM((NCHUNK, CH), jnp.int32),
            pltpu.VMEM((NCHUNK, CH), jnp.int32),
            pltpu.VMEM((2, CH, DQ), jnp.float32),
            pltpu.VMEM((ZROWS, DQ), jnp.float32),
            pltpu.VMEM_SHARED((ACC_R, DQ), jnp.float32),
            pltpu.SemaphoreType.DMA,
        ],
    )
    def k(h4_hbm, src_hbm, dst_hbm, out_hbm,
          src_v, dst_v, gidx_v, ldst_v, rows2, zbuf, acc, sem_a):
        _segsum_body(h4_hbm, src_hbm, dst_hbm, out_hbm,
                     src_v, dst_v, gidx_v, ldst_v, rows2, zbuf, sem_a, acc)

    return k


_segsum_sc = _build_segsum()


_BM = 1000


def _mm_init_kernel(x_ref, w_ref, o_ref):
    o_ref[...] = lax.dot_general(
        x_ref[...], w_ref[...], (((1,), (1,)), ((), ())),
        preferred_element_type=jnp.float32)


def _mm_init(x, W0):
    return pl.pallas_call(
        _mm_init_kernel,
        grid=(N // _BM,),
        in_specs=[
            pl.BlockSpec((_BM, F_IN), lambda i: (i, 0)),
            pl.BlockSpec((D, F_IN), lambda i: (0, 0)),
        ],
        out_specs=pl.BlockSpec((_BM, D), lambda i: (i, 0)),
        out_shape=jax.ShapeDtypeStruct((N, D), jnp.float32),
    )(x, W0)


def _layer_kernel(agg_ref, h_ref, w_ref, b_ref, wr_ref, br_ref, o_ref):
    z = lax.dot_general(
        agg_ref[0], w_ref[...][:, 0:DQ], (((1,), (1,)), ((), ())),
        preferred_element_type=jnp.float32)
    for q in range(1, Q):
        z = z + lax.dot_general(
            agg_ref[q], w_ref[...][:, q * DQ:(q + 1) * DQ],
            (((1,), (1,)), ((), ())), preferred_element_type=jnp.float32)
    r = lax.dot_general(
        h_ref[...], wr_ref[...], (((1,), (1,)), ((), ())),
        preferred_element_type=jnp.float32)
    o_ref[...] = (jnp.maximum(z + b_ref[...], 0.0)
                  + jnp.maximum(r + br_ref[...], 0.0))


def _layer(agg, h, W, b, Wr, br):
    return pl.pallas_call(
        _layer_kernel,
        grid=(N // _BM,),
        in_specs=[
            pl.BlockSpec((Q, _BM, DQ), lambda i: (0, i, 0)),
            pl.BlockSpec((_BM, D), lambda i: (i, 0)),
            pl.BlockSpec((D, D), lambda i: (0, 0)),
            pl.BlockSpec((1, D), lambda i: (0, 0)),
            pl.BlockSpec((D, D), lambda i: (0, 0)),
            pl.BlockSpec((1, D), lambda i: (0, 0)),
        ],
        out_specs=pl.BlockSpec((_BM, D), lambda i: (i, 0)),
        out_shape=jax.ShapeDtypeStruct((N, D), jnp.float32),
    )(agg, h, W, b, Wr, br)


@jax.jit
def _impl_a(x, edge_index, W0, W1, b1, Wr1, br1):
    pad = E_PAD - E
    src3 = jnp.concatenate(
        [edge_index[0], jnp.zeros((pad,), jnp.int32)]).reshape(NS, NCHUNK, CH)
    dst3 = jnp.concatenate(
        [edge_index[1],
         jnp.full((pad,), PAD_DST, jnp.int32)]).reshape(NS, NCHUNK, CH)
    h0 = _mm_init(x, W0)
    agg1 = _segsum_sc(h0.reshape(N * Q, DQ), src3, dst3)
    h1 = _layer(agg1, h0, W1, b1.reshape(1, D), Wr1, br1.reshape(1, D))
    return h1, src3, dst3


@jax.jit
def _impl_b(h1, src3, dst3, W2, b2, Wr2, br2):
    agg2 = _segsum_sc(h1.reshape(N * Q, DQ), src3, dst3)
    h2 = _layer(agg2, h1, W2, b2.reshape(1, D), Wr2, br2.reshape(1, D))
    return h2.reshape(BATCH, N // BATCH, D)


def kernel(x, edge_index, batch_size, W0, W1, b1, Wr1, br1, W2, b2, Wr2, br2):
    del batch_size
    h1, src3, dst3 = _impl_a(x, edge_index, W0, W1, b1, Wr1, br1)
    return _impl_b(h1, src3, dst3, W2, b2, Wr2, br2)


# speedup vs baseline: 1.7761x; 1.7761x over previous
import functools

import jax
import jax.numpy as jnp
from jax import lax
from jax.experimental import pallas as pl
from jax.experimental.pallas import tpu as pltpu
from jax.experimental.pallas import tpu_sc as plsc

N = 10000
E = 160000
F_IN = 256
D = 512
BATCH = 100

Q = 4
DQ = D // Q
NS = 16
NPASS = 3
NT = 3336
DUMP = NT
ACC_R = NT + 8
EPT = E // NS
CH = 80
NCHUNK = EPT // CH
NPT = 208
TAIL = NT - NS * NPT
ZROWS = 208

_VEC = 16


def _segsum_body(h4_hbm, src_hbm, dst_hbm, out_hbm,
                 src_v, dst_v, gidx_v, ldst_v, rows_v, zbuf, sem, acc):
    s = lax.axis_index("s")

    pltpu.sync_copy(src_hbm.at[s], src_v)
    pltpu.sync_copy(dst_hbm.at[s], dst_v)

    zeros = jnp.zeros((_VEC,), jnp.float32)

    def zfill(i, carry):
        for kk in range(DQ // _VEC):
            zbuf[i, pl.ds(kk * _VEC, _VEC)] = zeros
        return carry

    lax.fori_loop(0, ZROWS, zfill, 0)

    dump16 = jnp.full((_VEC,), DUMP, jnp.int32)

    for q in range(Q):
        def gidx(i, carry):
            for kk in range(CH // _VEC):
                sl = pl.ds(kk * _VEC, _VEC)
                gidx_v[i, sl] = src_v[i, sl] * Q + q
            return carry

        lax.fori_loop(0, NCHUNK, gidx, 0)

        for t in range(NPASS):
            lo = t * NT
            lo16 = jnp.full((_VEC,), lo, jnp.int32)
            hi16 = jnp.full((_VEC,), min(lo + NT, N), jnp.int32)

            def localize(i, carry):
                for kk in range(CH // _VEC):
                    sl = pl.ds(kk * _VEC, _VEC)
                    dv = dst_v[i, sl]
                    oob = (dv < lo16) | (dv >= hi16)
                    ldst_v[i, sl] = jnp.where(oob, dump16, dv - lo16)
                return carry

            lax.fori_loop(0, NCHUNK, localize, 0)

            pltpu.sync_copy(zbuf, acc.at[pl.ds(s * NPT, ZROWS)])

            if lo + NT <= N:
                @pl.when(s == NS - 1)
                def _():
                    pltpu.sync_copy(zbuf.at[pl.ds(0, TAIL)],
                                    acc.at[pl.ds(NS * NPT, TAIL)])

            plsc.subcore_barrier()

            def chunk(j, carry):
                pltpu.async_copy(h4_hbm.at[gidx_v.at[j]], rows_v,
                                 sem).wait()
                pltpu.sync_copy(rows_v, acc.at[ldst_v.at[j]], add=True)
                return carry

            lax.fori_loop(0, NCHUNK, chunk, 0)

            plsc.subcore_barrier()

            pltpu.sync_copy(
                acc.at[pl.ds(s * NPT, NPT)],
                out_hbm.at[q].at[pl.ds(lo + s * NPT, NPT)])

            if lo + NT <= N:
                @pl.when(s == NS - 1)
                def _():
                    pltpu.sync_copy(
                        acc.at[pl.ds(NS * NPT, TAIL)],
                        out_hbm.at[q].at[pl.ds(lo + NS * NPT, TAIL)])


def _build_segsum():
    mesh = plsc.VectorSubcoreMesh(core_axis_name="c", subcore_axis_name="s",
                                  num_cores=1)

    @functools.partial(
        pl.kernel,
        out_type=jax.ShapeDtypeStruct((Q, N, DQ), jnp.float32),
        mesh=mesh,
        scratch_types=[
            pltpu.VMEM((NCHUNK, CH), jnp.int32),
            pltpu.VMEM((NCHUNK, CH), jnp.int32),
            pltpu.VMEM((NCHUNK, CH), jnp.int32),
            pltpu.VMEM((NCHUNK, CH), jnp.int32),
            pltpu.VMEM((CH, DQ), jnp.float32),
            pltpu.VMEM((ZROWS, DQ), jnp.float32),
            pltpu.VMEM_SHARED((ACC_R, DQ), jnp.float32),
            pltpu.SemaphoreType.DMA,
        ],
    )
    def k(h4_hbm, src_hbm, dst_hbm, out_hbm,
          src_v, dst_v, gidx_v, ldst_v, rows_v, zbuf, acc, sem):
        _segsum_body(h4_hbm, src_hbm, dst_hbm, out_hbm,
                     src_v, dst_v, gidx_v, ldst_v, rows_v, zbuf, sem, acc)

    return k


_segsum_sc = _build_segsum()


_BM = 1000


def _mm_init_kernel(x_ref, w_ref, o_ref):
    o_ref[...] = lax.dot_general(
        x_ref[...], w_ref[...], (((1,), (1,)), ((), ())),
        preferred_element_type=jnp.float32)


def _mm_init(x, W0):
    return pl.pallas_call(
        _mm_init_kernel,
        grid=(N // _BM,),
        in_specs=[
            pl.BlockSpec((_BM, F_IN), lambda i: (i, 0)),
            pl.BlockSpec((D, F_IN), lambda i: (0, 0)),
        ],
        out_specs=pl.BlockSpec((_BM, D), lambda i: (i, 0)),
        out_shape=jax.ShapeDtypeStruct((N, D), jnp.float32),
    )(x, W0)


def _layer_kernel(agg_ref, h_ref, w_ref, b_ref, wr_ref, br_ref, o_ref):
    z = lax.dot_general(
        agg_ref[0], w_ref[...][:, 0:DQ], (((1,), (1,)), ((), ())),
        preferred_element_type=jnp.float32)
    for q in range(1, Q):
        z = z + lax.dot_general(
            agg_ref[q], w_ref[...][:, q * DQ:(q + 1) * DQ],
            (((1,), (1,)), ((), ())), preferred_element_type=jnp.float32)
    r = lax.dot_general(
        h_ref[...], wr_ref[...], (((1,), (1,)), ((), ())),
        preferred_element_type=jnp.float32)
    o_ref[...] = (jnp.maximum(z + b_ref[...], 0.0)
                  + jnp.maximum(r + br_ref[...], 0.0))


def _layer(agg, h, W, b, Wr, br):
    return pl.pallas_call(
        _layer_kernel,
        grid=(N // _BM,),
        in_specs=[
            pl.BlockSpec((Q, _BM, DQ), lambda i: (0, i, 0)),
            pl.BlockSpec((_BM, D), lambda i: (i, 0)),
            pl.BlockSpec((D, D), lambda i: (0, 0)),
            pl.BlockSpec((1, D), lambda i: (0, 0)),
            pl.BlockSpec((D, D), lambda i: (0, 0)),
            pl.BlockSpec((1, D), lambda i: (0, 0)),
        ],
        out_specs=pl.BlockSpec((_BM, D), lambda i: (i, 0)),
        out_shape=jax.ShapeDtypeStruct((N, D), jnp.float32),
    )(agg, h, W, b, Wr, br)


@jax.jit
def _impl_a(x, edge_index, W0, W1, b1, Wr1, br1):
    src3 = edge_index[0].reshape(NS, NCHUNK, CH)
    dst3 = edge_index[1].reshape(NS, NCHUNK, CH)
    h0 = _mm_init(x, W0)
    agg1 = _segsum_sc(h0.reshape(N * Q, DQ), src3, dst3)
    h1 = _layer(agg1, h0, W1, b1.reshape(1, D), Wr1, br1.reshape(1, D))
    return h1, src3, dst3


@jax.jit
def _impl_b(h1, src3, dst3, W2, b2, Wr2, br2):
    agg2 = _segsum_sc(h1.reshape(N * Q, DQ), src3, dst3)
    h2 = _layer(agg2, h1, W2, b2.reshape(1, D), Wr2, br2.reshape(1, D))
    return h2.reshape(BATCH, N // BATCH, D)


def kernel(x, edge_index, batch_size, W0, W1, b1, Wr1, br1, W2, b2, Wr2, br2):
    del batch_size
    h1, src3, dst3 = _impl_a(x, edge_index, W0, W1, b1, Wr1, br1)
    return _impl_b(h1, src3, dst3, W2, b2, Wr2, br2)
